# initial kernel scaffold (unmeasured)
import jax
import jax.numpy as jnp
from jax import lax
from jax.experimental import pallas as pl
from jax.experimental.pallas import tpu as pltpu

N_DEV = 32
B, SQ, SKV, DM = 2, 128, 128, 512
HL, DH = 4, 64
ROWS = B * SQ
CHUNK = ROWS // N_DEV


def kernel(x, Wq, K_ext, V_ext, Wo):
    x2 = x.reshape(ROWS, DM)

    def body(x_ref, wq_ref, k_hbm, v_hbm, wo_ref, out_ref,
             kbuf, vbuf, cbuf, pbuf, p1buf, gbuf,
             local_sems, send1, recv1, send2, recv2):
        me = lax.axis_index("i")

        h0 = me * HL
        kcp = pltpu.make_async_copy(
            k_hbm.at[:, :, pl.ds(h0, HL), :], kbuf, local_sems.at[0])
        vcp = pltpu.make_async_copy(
            v_hbm.at[:, :, pl.ds(h0, HL), :], vbuf, local_sems.at[1])
        kcp.start()
        vcp.start()

        q = lax.dot_general(
            x_ref[...].astype(jnp.bfloat16), wq_ref[...].astype(jnp.bfloat16),
            (((1,), (0,)), ((), ())), preferred_element_type=jnp.float32)

        kcp.wait()
        vcp.wait()
        kv_k = kbuf[...]
        kv_v = vbuf[...]

        for b in range(B):
            for h in range(HL):
                qbh = q[b * SQ:(b + 1) * SQ, h * DH:(h + 1) * DH]
                kbh = kv_k[b, :, h, :]
                s = lax.dot_general(
                    qbh.astype(jnp.bfloat16), kbh.astype(jnp.bfloat16),
                    (((1,), (1,)), ((), ())),
                    preferred_element_type=jnp.float32) * 0.125
                mx = jnp.max(s, axis=1, keepdims=True)
                w = jnp.exp(s - mx)
                w = w / jnp.sum(w, axis=1, keepdims=True)
                c = lax.dot_general(
                    w.astype(jnp.bfloat16), kv_v[b, :, h, :].astype(jnp.bfloat16),
                    (((1,), (0,)), ((), ())),
                    preferred_element_type=jnp.float32)
                cbuf[b * SQ:(b + 1) * SQ, h * DH:(h + 1) * DH] = c

        pbuf[...] = lax.dot_general(
            cbuf[...].astype(jnp.bfloat16), wo_ref[...].astype(jnp.bfloat16),
            (((1,), (0,)), ((), ())), preferred_element_type=jnp.float32)

        bsem = pltpu.get_barrier_semaphore()
        for off in range(1, N_DEV):
            pl.semaphore_signal(
                bsem, inc=1, device_id=((me + off) % N_DEV,),
                device_id_type=pl.DeviceIdType.MESH)
        pl.semaphore_wait(bsem, N_DEV - 1)

        sends1 = []
        for off in range(1, N_DEV):
            dst = (me + off) % N_DEV
            r = pltpu.make_async_remote_copy(
                src_ref=pbuf.at[pl.ds(dst * CHUNK, CHUNK), :],
                dst_ref=p1buf.at[me],
                send_sem=send1.at[off - 1],
                recv_sem=recv1.at[me],
                device_id=(dst,), device_id_type=pl.DeviceIdType.MESH)
            r.start()
            sends1.append(r)

        acc = pbuf[pl.ds(me * CHUNK, CHUNK), :]
        for off in range(1, N_DEV):
            src = (me + N_DEV - off) % N_DEV
            rv = pltpu.make_async_remote_copy(
                src_ref=pbuf.at[pl.ds(0, CHUNK), :],
                dst_ref=p1buf.at[src],
                send_sem=send1.at[off - 1],
                recv_sem=recv1.at[src],
                device_id=(src,), device_id_type=pl.DeviceIdType.MESH)
            rv.wait_recv()
            acc = acc + p1buf[src]

        gbuf[pl.ds(me * CHUNK, CHUNK), :] = acc

        sends2 = []
        for off in range(1, N_DEV):
            dst = (me + off) % N_DEV
            r = pltpu.make_async_remote_copy(
                src_ref=gbuf.at[pl.ds(me * CHUNK, CHUNK), :],
                dst_ref=gbuf.at[pl.ds(me * CHUNK, CHUNK), :],
                send_sem=send2.at[off - 1],
                recv_sem=recv2.at[me],
                device_id=(dst,), device_id_type=pl.DeviceIdType.MESH)
            r.start()
            sends2.append(r)

        for off in range(1, N_DEV):
            src = (me + N_DEV - off) % N_DEV
            rv = pltpu.make_async_remote_copy(
                src_ref=gbuf.at[pl.ds(0, CHUNK), :],
                dst_ref=gbuf.at[pl.ds(src * CHUNK, CHUNK), :],
                send_sem=send2.at[off - 1],
                recv_sem=recv2.at[src],
                device_id=(src,), device_id_type=pl.DeviceIdType.MESH)
            rv.wait_recv()

        for r in sends1 + sends2:
            r.wait_send()

        out_ref[...] = gbuf[...]

    out = pl.pallas_call(
        body,
        out_shape=jax.ShapeDtypeStruct((ROWS, DM), jnp.float32),
        in_specs=[
            pl.BlockSpec(memory_space=pltpu.VMEM),
            pl.BlockSpec(memory_space=pltpu.VMEM),
            pl.BlockSpec(memory_space=pltpu.ANY),
            pl.BlockSpec(memory_space=pltpu.ANY),
            pl.BlockSpec(memory_space=pltpu.VMEM),
        ],
        out_specs=pl.BlockSpec(memory_space=pltpu.VMEM),
        scratch_shapes=[
            pltpu.VMEM((B, SKV, HL, DH), jnp.float32),
            pltpu.VMEM((B, SKV, HL, DH), jnp.float32),
            pltpu.VMEM((ROWS, HL * DH), jnp.float32),
            pltpu.VMEM((ROWS, DM), jnp.float32),
            pltpu.VMEM((N_DEV, CHUNK, DM), jnp.float32),
            pltpu.VMEM((ROWS, DM), jnp.float32),
            pltpu.SemaphoreType.DMA((2,)),
            pltpu.SemaphoreType.DMA((N_DEV - 1,)),
            pltpu.SemaphoreType.DMA((N_DEV,)),
            pltpu.SemaphoreType.DMA((N_DEV - 1,)),
            pltpu.SemaphoreType.DMA((N_DEV,)),
        ],
        compiler_params=pltpu.CompilerParams(collective_id=0),
    )(x2, Wq, K_ext, V_ext, Wo)

    return out.reshape(B, SQ, DM)


# baseline (device time: 47294 ns/iter reference)
import jax
import jax.numpy as jnp
from jax import lax
from jax.experimental import pallas as pl
from jax.experimental.pallas import tpu as pltpu

N_DEV = 32
B, SQ, SKV, DM = 2, 128, 128, 512
HL, DH = 4, 64
ROWS = B * SQ
CHUNK = ROWS // N_DEV


def kernel(x, Wq, K_ext, V_ext, Wo):
    x2 = x.reshape(ROWS, DM)

    def body(x_ref, wq_ref, k_hbm, v_hbm, wo_ref, out_ref,
             kbuf, vbuf, cbuf, pbuf, p1buf, gbuf,
             local_sems, send1, recv1, send2, recv2):
        me = lax.axis_index("i")

        h0 = me * HL
        kcp = pltpu.make_async_copy(
            k_hbm.at[:, :, pl.ds(h0, HL), :], kbuf, local_sems.at[0])
        vcp = pltpu.make_async_copy(
            v_hbm.at[:, :, pl.ds(h0, HL), :], vbuf, local_sems.at[1])
        kcp.start()
        vcp.start()

        q = lax.dot_general(
            x_ref[...].astype(jnp.bfloat16), wq_ref[...].astype(jnp.bfloat16),
            (((1,), (0,)), ((), ())), preferred_element_type=jnp.float32)

        kcp.wait()
        vcp.wait()
        kv_k = kbuf[...]
        kv_v = vbuf[...]

        for b in range(B):
            for h in range(HL):
                qbh = q[b * SQ:(b + 1) * SQ, h * DH:(h + 1) * DH]
                kbh = kv_k[b, :, h, :]
                s = lax.dot_general(
                    qbh.astype(jnp.bfloat16), kbh.astype(jnp.bfloat16),
                    (((1,), (1,)), ((), ())),
                    preferred_element_type=jnp.float32) * 0.125
                mx = jnp.max(s, axis=1, keepdims=True)
                w = jnp.exp(s - mx)
                w = w / jnp.sum(w, axis=1, keepdims=True)
                c = lax.dot_general(
                    w.astype(jnp.bfloat16), kv_v[b, :, h, :].astype(jnp.bfloat16),
                    (((1,), (0,)), ((), ())),
                    preferred_element_type=jnp.float32)
                cbuf[b * SQ:(b + 1) * SQ, h * DH:(h + 1) * DH] = c

        pbuf[...] = lax.dot_general(
            cbuf[...].astype(jnp.bfloat16), wo_ref[...].astype(jnp.bfloat16),
            (((1,), (0,)), ((), ())), preferred_element_type=jnp.float32)

        bsem = pltpu.get_barrier_semaphore()
        for off in range(1, N_DEV):
            pl.semaphore_signal(
                bsem, inc=1, device_id=((me + off) % N_DEV,),
                device_id_type=pl.DeviceIdType.MESH)
        pl.semaphore_wait(bsem, N_DEV - 1)

        sends1 = []
        for off in range(1, N_DEV):
            dst = (me + off) % N_DEV
            r = pltpu.make_async_remote_copy(
                src_ref=pbuf.at[pl.ds(dst * CHUNK, CHUNK), :],
                dst_ref=p1buf.at[me],
                send_sem=send1.at[off - 1],
                recv_sem=recv1.at[me],
                device_id=(dst,), device_id_type=pl.DeviceIdType.MESH)
            r.start()
            sends1.append(r)

        acc = pbuf[pl.ds(me * CHUNK, CHUNK), :]
        for off in range(1, N_DEV):
            src = (me + N_DEV - off) % N_DEV
            rv = pltpu.make_async_remote_copy(
                src_ref=pbuf.at[pl.ds(0, CHUNK), :],
                dst_ref=p1buf.at[src],
                send_sem=send1.at[off - 1],
                recv_sem=recv1.at[src],
                device_id=(src,), device_id_type=pl.DeviceIdType.MESH)
            rv.wait_recv()
            acc = acc + p1buf[src]

        gbuf[pl.ds(me * CHUNK, CHUNK), :] = acc

        sends2 = []
        for off in range(1, N_DEV):
            dst = (me + off) % N_DEV
            r = pltpu.make_async_remote_copy(
                src_ref=gbuf.at[pl.ds(me * CHUNK, CHUNK), :],
                dst_ref=gbuf.at[pl.ds(me * CHUNK, CHUNK), :],
                send_sem=send2.at[off - 1],
                recv_sem=recv2.at[me],
                device_id=(dst,), device_id_type=pl.DeviceIdType.MESH)
            r.start()
            sends2.append(r)

        for off in range(1, N_DEV):
            src = (me + N_DEV - off) % N_DEV
            rv = pltpu.make_async_remote_copy(
                src_ref=gbuf.at[pl.ds(0, CHUNK), :],
                dst_ref=gbuf.at[pl.ds(src * CHUNK, CHUNK), :],
                send_sem=send2.at[off - 1],
                recv_sem=recv2.at[src],
                device_id=(src,), device_id_type=pl.DeviceIdType.MESH)
            rv.wait_recv()

        for r in sends1 + sends2:
            r.wait_send()

        out_ref[...] = gbuf[...]

    out = pl.pallas_call(
        body,
        out_shape=jax.ShapeDtypeStruct((ROWS, DM), jnp.float32),
        in_specs=[
            pl.BlockSpec(memory_space=pltpu.VMEM),
            pl.BlockSpec(memory_space=pltpu.VMEM),
            pl.BlockSpec(memory_space=pl.ANY),
            pl.BlockSpec(memory_space=pl.ANY),
            pl.BlockSpec(memory_space=pltpu.VMEM),
        ],
        out_specs=pl.BlockSpec(memory_space=pltpu.VMEM),
        scratch_shapes=[
            pltpu.VMEM((B, SKV, HL, DH), jnp.float32),
            pltpu.VMEM((B, SKV, HL, DH), jnp.float32),
            pltpu.VMEM((ROWS, HL * DH), jnp.float32),
            pltpu.VMEM((ROWS, DM), jnp.float32),
            pltpu.VMEM((N_DEV, CHUNK, DM), jnp.float32),
            pltpu.VMEM((ROWS, DM), jnp.float32),
            pltpu.SemaphoreType.DMA((2,)),
            pltpu.SemaphoreType.DMA((N_DEV - 1,)),
            pltpu.SemaphoreType.DMA((N_DEV,)),
            pltpu.SemaphoreType.DMA((N_DEV - 1,)),
            pltpu.SemaphoreType.DMA((N_DEV,)),
        ],
        compiler_params=pltpu.CompilerParams(collective_id=0),
    )(x2, Wq, K_ext, V_ext, Wo)

    return out.reshape(B, SQ, DM)


# device time: 43593 ns/iter; 1.0849x vs baseline; 1.0849x over previous
import jax
import jax.numpy as jnp
from jax import lax
from jax.experimental import pallas as pl
from jax.experimental.pallas import tpu as pltpu

N_DEV = 32
B, SQ, SKV, DM = 2, 128, 128, 512
HL, DH = 4, 64
ROWS = B * SQ
CHUNK = ROWS // N_DEV


def kernel(x, Wq, K_ext, V_ext, Wo):
    x2 = x.reshape(ROWS, DM)

    def body(x_ref, wq_ref, k_hbm, v_hbm, wo_ref, out_ref,
             kbuf, vbuf, cbuf, pbuf, p1buf, gbuf,
             local_sems, send1, recv1, send2, recv2):
        me = lax.axis_index("i")

        h0 = me * HL
        kcp = pltpu.make_async_copy(
            k_hbm.at[:, :, pl.ds(h0, HL), :], kbuf, local_sems.at[0])
        vcp = pltpu.make_async_copy(
            v_hbm.at[:, :, pl.ds(h0, HL), :], vbuf, local_sems.at[1])
        kcp.start()
        vcp.start()

        q = lax.dot_general(
            x_ref[...].astype(jnp.bfloat16), wq_ref[...].astype(jnp.bfloat16),
            (((1,), (0,)), ((), ())), preferred_element_type=jnp.float32)

        kcp.wait()
        vcp.wait()
        kv_k = kbuf[...]
        kv_v = vbuf[...]

        for b in range(B):
            for h in range(HL):
                qbh = q[b * SQ:(b + 1) * SQ, h * DH:(h + 1) * DH]
                kbh = kv_k[b, :, h, :]
                s = lax.dot_general(
                    qbh.astype(jnp.bfloat16), kbh.astype(jnp.bfloat16),
                    (((1,), (1,)), ((), ())),
                    preferred_element_type=jnp.float32) * 0.125
                mx = jnp.max(s, axis=1, keepdims=True)
                w = jnp.exp(s - mx)
                w = w / jnp.sum(w, axis=1, keepdims=True)
                c = lax.dot_general(
                    w.astype(jnp.bfloat16), kv_v[b, :, h, :].astype(jnp.bfloat16),
                    (((1,), (0,)), ((), ())),
                    preferred_element_type=jnp.float32)
                cbuf[b * SQ:(b + 1) * SQ, h * DH:(h + 1) * DH] = c

        partial = lax.dot_general(
            cbuf[...].astype(jnp.bfloat16), wo_ref[...].astype(jnp.bfloat16),
            (((1,), (0,)), ((), ())), preferred_element_type=jnp.float32)
        pbuf[...] = partial.reshape(N_DEV, CHUNK, DM).astype(jnp.bfloat16)

        bsem = pltpu.get_barrier_semaphore()
        for off in range(1, N_DEV):
            pl.semaphore_signal(
                bsem, inc=1, device_id=((me + off) % N_DEV,),
                device_id_type=pl.DeviceIdType.MESH)
        pl.semaphore_wait(bsem, N_DEV - 1)

        sends1 = []
        for off in range(1, N_DEV):
            dst = (me + off) % N_DEV
            r = pltpu.make_async_remote_copy(
                src_ref=pbuf.at[dst],
                dst_ref=p1buf.at[me],
                send_sem=send1.at[off - 1],
                recv_sem=recv1.at[me],
                device_id=(dst,), device_id_type=pl.DeviceIdType.MESH)
            r.start()
            sends1.append(r)

        acc = pbuf[me].astype(jnp.float32)
        for off in range(1, N_DEV):
            src = (me + N_DEV - off) % N_DEV
            rv = pltpu.make_async_remote_copy(
                src_ref=pbuf.at[0],
                dst_ref=p1buf.at[src],
                send_sem=send1.at[off - 1],
                recv_sem=recv1.at[src],
                device_id=(src,), device_id_type=pl.DeviceIdType.MESH)
            rv.wait_recv()
            acc = acc + p1buf[src].astype(jnp.float32)

        gbuf[me] = acc.astype(jnp.bfloat16)

        sends2 = []
        for off in range(1, N_DEV):
            dst = (me + off) % N_DEV
            r = pltpu.make_async_remote_copy(
                src_ref=gbuf.at[me],
                dst_ref=gbuf.at[me],
                send_sem=send2.at[off - 1],
                recv_sem=recv2.at[me],
                device_id=(dst,), device_id_type=pl.DeviceIdType.MESH)
            r.start()
            sends2.append(r)

        for off in range(1, N_DEV):
            src = (me + N_DEV - off) % N_DEV
            rv = pltpu.make_async_remote_copy(
                src_ref=gbuf.at[0],
                dst_ref=gbuf.at[src],
                send_sem=send2.at[off - 1],
                recv_sem=recv2.at[src],
                device_id=(src,), device_id_type=pl.DeviceIdType.MESH)
            rv.wait_recv()

        for r in sends1 + sends2:
            r.wait_send()

        out_ref[...] = gbuf[...].astype(jnp.float32).reshape(ROWS, DM)

    out = pl.pallas_call(
        body,
        out_shape=jax.ShapeDtypeStruct((ROWS, DM), jnp.float32),
        in_specs=[
            pl.BlockSpec(memory_space=pltpu.VMEM),
            pl.BlockSpec(memory_space=pltpu.VMEM),
            pl.BlockSpec(memory_space=pl.ANY),
            pl.BlockSpec(memory_space=pl.ANY),
            pl.BlockSpec(memory_space=pltpu.VMEM),
        ],
        out_specs=pl.BlockSpec(memory_space=pltpu.VMEM),
        scratch_shapes=[
            pltpu.VMEM((B, SKV, HL, DH), jnp.float32),
            pltpu.VMEM((B, SKV, HL, DH), jnp.float32),
            pltpu.VMEM((ROWS, HL * DH), jnp.float32),
            pltpu.VMEM((N_DEV, CHUNK, DM), jnp.bfloat16),
            pltpu.VMEM((N_DEV, CHUNK, DM), jnp.bfloat16),
            pltpu.VMEM((N_DEV, CHUNK, DM), jnp.bfloat16),
            pltpu.SemaphoreType.DMA((2,)),
            pltpu.SemaphoreType.DMA((N_DEV - 1,)),
            pltpu.SemaphoreType.DMA((N_DEV,)),
            pltpu.SemaphoreType.DMA((N_DEV - 1,)),
            pltpu.SemaphoreType.DMA((N_DEV,)),
        ],
        compiler_params=pltpu.CompilerParams(collective_id=0),
    )(x2, Wq, K_ext, V_ext, Wo)

    return out.reshape(B, SQ, DM)


# device time: 23873 ns/iter; 1.9811x vs baseline; 1.8260x over previous
import os

import jax
import jax.numpy as jnp
from jax import lax
from jax.experimental import pallas as pl
from jax.experimental.pallas import tpu as pltpu

_ABLATE = os.environ.get("KERNEL_ABLATE", "full")

N_DEV = 32
B, SQ, SKV, DM = 2, 128, 128, 512
HL, DH = 4, 64
ROWS = B * SQ
CHUNK = ROWS // N_DEV


def kernel(x, Wq, K_ext, V_ext, Wo):
    x2 = x.reshape(ROWS, DM)

    def body(x_ref, wq_ref, k_hbm, v_hbm, wo_ref, out_ref,
             kbuf, vbuf, cbuf, pbuf, p1buf, gbuf,
             local_sems, send1, recv1, send2, recv2):
        me = lax.axis_index("i")

        h0 = me * HL
        kcp = pltpu.make_async_copy(
            k_hbm.at[:, :, pl.ds(h0, HL), :], kbuf, local_sems.at[0])
        vcp = pltpu.make_async_copy(
            v_hbm.at[:, :, pl.ds(h0, HL), :], vbuf, local_sems.at[1])
        kcp.start()
        vcp.start()

        q = lax.dot_general(
            x_ref[...].astype(jnp.bfloat16), wq_ref[...].astype(jnp.bfloat16),
            (((1,), (0,)), ((), ())), preferred_element_type=jnp.float32)

        kcp.wait()
        vcp.wait()
        kv_k = kbuf[...]
        kv_v = vbuf[...]

        for b in range(B):
            for h in range(HL):
                qbh = q[b * SQ:(b + 1) * SQ, h * DH:(h + 1) * DH]
                kbh = kv_k[b, :, h, :]
                s = lax.dot_general(
                    qbh.astype(jnp.bfloat16), kbh.astype(jnp.bfloat16),
                    (((1,), (1,)), ((), ())),
                    preferred_element_type=jnp.float32) * 0.125
                mx = jnp.max(s, axis=1, keepdims=True)
                w = jnp.exp(s - mx)
                w = w / jnp.sum(w, axis=1, keepdims=True)
                c = lax.dot_general(
                    w.astype(jnp.bfloat16), kv_v[b, :, h, :].astype(jnp.bfloat16),
                    (((1,), (0,)), ((), ())),
                    preferred_element_type=jnp.float32)
                cbuf[b * SQ:(b + 1) * SQ, h * DH:(h + 1) * DH] = c

        partial = lax.dot_general(
            cbuf[...].astype(jnp.bfloat16), wo_ref[...].astype(jnp.bfloat16),
            (((1,), (0,)), ((), ())), preferred_element_type=jnp.float32)
        pbuf[...] = partial.reshape(N_DEV, CHUNK, DM).astype(jnp.bfloat16)

        if _ABLATE == "compute":
            out_ref[...] = pbuf[...].astype(jnp.float32).reshape(ROWS, DM)
            return

        bsem = pltpu.get_barrier_semaphore()
        for off in range(1, N_DEV):
            pl.semaphore_signal(
                bsem, inc=1, device_id=((me + off) % N_DEV,),
                device_id_type=pl.DeviceIdType.MESH)
        pl.semaphore_wait(bsem, N_DEV - 1)

        sends1 = []
        for off in range(1, N_DEV):
            dst = (me + off) % N_DEV
            r = pltpu.make_async_remote_copy(
                src_ref=pbuf.at[dst],
                dst_ref=p1buf.at[me],
                send_sem=send1.at[off - 1],
                recv_sem=recv1.at[me],
                device_id=(dst,), device_id_type=pl.DeviceIdType.MESH)
            r.start()
            sends1.append(r)

        acc = pbuf[me].astype(jnp.float32)
        for off in range(1, N_DEV):
            src = (me + N_DEV - off) % N_DEV
            rv = pltpu.make_async_remote_copy(
                src_ref=pbuf.at[0],
                dst_ref=p1buf.at[src],
                send_sem=send1.at[off - 1],
                recv_sem=recv1.at[src],
                device_id=(src,), device_id_type=pl.DeviceIdType.MESH)
            rv.wait_recv()
            acc = acc + p1buf[src].astype(jnp.float32)

        gbuf[me] = acc.astype(jnp.bfloat16)

        if _ABLATE == "phase1":
            for r in sends1:
                r.wait_send()
            out_ref[...] = gbuf[...].astype(jnp.float32).reshape(ROWS, DM)
            return

        sends2 = []
        for off in range(1, N_DEV):
            dst = (me + off) % N_DEV
            r = pltpu.make_async_remote_copy(
                src_ref=gbuf.at[me],
                dst_ref=gbuf.at[me],
                send_sem=send2.at[off - 1],
                recv_sem=recv2.at[me],
                device_id=(dst,), device_id_type=pl.DeviceIdType.MESH)
            r.start()
            sends2.append(r)

        for off in range(1, N_DEV):
            src = (me + N_DEV - off) % N_DEV
            rv = pltpu.make_async_remote_copy(
                src_ref=gbuf.at[0],
                dst_ref=gbuf.at[src],
                send_sem=send2.at[off - 1],
                recv_sem=recv2.at[src],
                device_id=(src,), device_id_type=pl.DeviceIdType.MESH)
            rv.wait_recv()

        for r in sends1 + sends2:
            r.wait_send()

        out_ref[...] = gbuf[...].astype(jnp.float32).reshape(ROWS, DM)

    out = pl.pallas_call(
        body,
        out_shape=jax.ShapeDtypeStruct((ROWS, DM), jnp.float32),
        in_specs=[
            pl.BlockSpec(memory_space=pltpu.VMEM),
            pl.BlockSpec(memory_space=pltpu.VMEM),
            pl.BlockSpec(memory_space=pl.ANY),
            pl.BlockSpec(memory_space=pl.ANY),
            pl.BlockSpec(memory_space=pltpu.VMEM),
        ],
        out_specs=pl.BlockSpec(memory_space=pltpu.VMEM),
        scratch_shapes=[
            pltpu.VMEM((B, SKV, HL, DH), jnp.float32),
            pltpu.VMEM((B, SKV, HL, DH), jnp.float32),
            pltpu.VMEM((ROWS, HL * DH), jnp.float32),
            pltpu.VMEM((N_DEV, CHUNK, DM), jnp.bfloat16),
            pltpu.VMEM((N_DEV, CHUNK, DM), jnp.bfloat16),
            pltpu.VMEM((N_DEV, CHUNK, DM), jnp.bfloat16),
            pltpu.SemaphoreType.DMA((2,)),
            pltpu.SemaphoreType.DMA((N_DEV - 1,)),
            pltpu.SemaphoreType.DMA((N_DEV,)),
            pltpu.SemaphoreType.DMA((N_DEV - 1,)),
            pltpu.SemaphoreType.DMA((N_DEV,)),
        ],
        compiler_params=pltpu.CompilerParams(
            collective_id=None if _ABLATE == "compute" else 0),
    )(x2, Wq, K_ext, V_ext, Wo)

    return out.reshape(B, SQ, DM)


# device time: 20517 ns/iter; 2.3051x vs baseline; 1.1636x over previous
import os

import jax
import jax.numpy as jnp
from jax import lax
from jax.experimental import pallas as pl
from jax.experimental.pallas import tpu as pltpu

_ABLATE = os.environ.get("KERNEL_ABLATE", "full")

N_DEV = 32
B, SQ, SKV, DM = 2, 128, 128, 512
HL, DH = 4, 64
ROWS = B * SQ
CHUNK = ROWS // N_DEV


def kernel(x, Wq, K_ext, V_ext, Wo):
    x2 = x.reshape(ROWS, DM)

    def body(x_ref, wq_ref, k_hbm, v_hbm, wo_ref, out_ref,
             kbuf, vbuf, cbuf, pbuf, p1buf, gbuf,
             local_sems, send1, recv1, send2, recv2):
        me = lax.axis_index("i")

        h0 = me * HL
        if _ABLATE not in ("nodma", "qproj"):
            kcp = pltpu.make_async_copy(
                k_hbm.at[:, :, pl.ds(h0, HL), :], kbuf, local_sems.at[0])
            vcp = pltpu.make_async_copy(
                v_hbm.at[:, :, pl.ds(h0, HL), :], vbuf, local_sems.at[1])
            kcp.start()
            vcp.start()

        q = lax.dot_general(
            x_ref[...].astype(jnp.bfloat16), wq_ref[...].astype(jnp.bfloat16),
            (((1,), (0,)), ((), ())), preferred_element_type=jnp.float32)

        if _ABLATE not in ("nodma", "qproj"):
            kcp.wait()
            vcp.wait()
        kv_k = kbuf[...]
        kv_v = vbuf[...]

        for b in range(B if _ABLATE != "qproj" else 0):
            for h in range(HL):
                qbh = q[b * SQ:(b + 1) * SQ, h * DH:(h + 1) * DH]
                kbh = kv_k[b, :, h, :]
                s = lax.dot_general(
                    qbh.astype(jnp.bfloat16), kbh.astype(jnp.bfloat16),
                    (((1,), (1,)), ((), ())),
                    preferred_element_type=jnp.float32) * 0.125
                mx = jnp.max(s, axis=1, keepdims=True)
                w = jnp.exp(s - mx)
                w = w / jnp.sum(w, axis=1, keepdims=True)
                c = lax.dot_general(
                    w.astype(jnp.bfloat16), kv_v[b, :, h, :].astype(jnp.bfloat16),
                    (((1,), (0,)), ((), ())),
                    preferred_element_type=jnp.float32)
                cbuf[b * SQ:(b + 1) * SQ, h * DH:(h + 1) * DH] = c

        ctx = q if _ABLATE == "qproj" else cbuf[...]
        partial = lax.dot_general(
            ctx.astype(jnp.bfloat16), wo_ref[...].astype(jnp.bfloat16),
            (((1,), (0,)), ((), ())), preferred_element_type=jnp.float32)
        pbuf[...] = partial.reshape(N_DEV, CHUNK, DM).astype(jnp.bfloat16)

        if _ABLATE in ("compute", "nodma", "qproj"):
            out_ref[...] = pbuf[...].astype(jnp.float32).reshape(ROWS, DM)
            return

        bsem = pltpu.get_barrier_semaphore()
        for off in range(1, N_DEV):
            pl.semaphore_signal(
                bsem, inc=1, device_id=((me + off) % N_DEV,),
                device_id_type=pl.DeviceIdType.MESH)
        pl.semaphore_wait(bsem, N_DEV - 1)

        sends1 = []
        for off in range(1, N_DEV):
            dst = (me + off) % N_DEV
            r = pltpu.make_async_remote_copy(
                src_ref=pbuf.at[dst],
                dst_ref=p1buf.at[me],
                send_sem=send1.at[off - 1],
                recv_sem=recv1.at[me],
                device_id=(dst,), device_id_type=pl.DeviceIdType.MESH)
            r.start()
            sends1.append(r)

        acc = pbuf[me].astype(jnp.float32)
        for off in range(1, N_DEV):
            src = (me + N_DEV - off) % N_DEV
            rv = pltpu.make_async_remote_copy(
                src_ref=pbuf.at[0],
                dst_ref=p1buf.at[src],
                send_sem=send1.at[off - 1],
                recv_sem=recv1.at[src],
                device_id=(src,), device_id_type=pl.DeviceIdType.MESH)
            rv.wait_recv()
            acc = acc + p1buf[src].astype(jnp.float32)

        gbuf[me] = acc.astype(jnp.bfloat16)

        if _ABLATE == "phase1":
            for r in sends1:
                r.wait_send()
            out_ref[...] = gbuf[...].astype(jnp.float32).reshape(ROWS, DM)
            return

        sends2 = []
        for off in range(1, N_DEV):
            dst = (me + off) % N_DEV
            r = pltpu.make_async_remote_copy(
                src_ref=gbuf.at[me],
                dst_ref=gbuf.at[me],
                send_sem=send2.at[off - 1],
                recv_sem=recv2.at[me],
                device_id=(dst,), device_id_type=pl.DeviceIdType.MESH)
            r.start()
            sends2.append(r)

        for off in range(1, N_DEV):
            src = (me + N_DEV - off) % N_DEV
            rv = pltpu.make_async_remote_copy(
                src_ref=gbuf.at[0],
                dst_ref=gbuf.at[src],
                send_sem=send2.at[off - 1],
                recv_sem=recv2.at[src],
                device_id=(src,), device_id_type=pl.DeviceIdType.MESH)
            rv.wait_recv()

        for r in sends1 + sends2:
            r.wait_send()

        out_ref[...] = gbuf[...].astype(jnp.float32).reshape(ROWS, DM)

    out = pl.pallas_call(
        body,
        out_shape=jax.ShapeDtypeStruct((ROWS, DM), jnp.float32),
        in_specs=[
            pl.BlockSpec(memory_space=pltpu.VMEM),
            pl.BlockSpec(memory_space=pltpu.VMEM),
            pl.BlockSpec(memory_space=pl.ANY),
            pl.BlockSpec(memory_space=pl.ANY),
            pl.BlockSpec(memory_space=pltpu.VMEM),
        ],
        out_specs=pl.BlockSpec(memory_space=pltpu.VMEM),
        scratch_shapes=[
            pltpu.VMEM((B, SKV, HL, DH), jnp.float32),
            pltpu.VMEM((B, SKV, HL, DH), jnp.float32),
            pltpu.VMEM((ROWS, HL * DH), jnp.float32),
            pltpu.VMEM((N_DEV, CHUNK, DM), jnp.bfloat16),
            pltpu.VMEM((N_DEV, CHUNK, DM), jnp.bfloat16),
            pltpu.VMEM((N_DEV, CHUNK, DM), jnp.bfloat16),
            pltpu.SemaphoreType.DMA((2,)),
            pltpu.SemaphoreType.DMA((N_DEV - 1,)),
            pltpu.SemaphoreType.DMA((N_DEV,)),
            pltpu.SemaphoreType.DMA((N_DEV - 1,)),
            pltpu.SemaphoreType.DMA((N_DEV,)),
        ],
        compiler_params=pltpu.CompilerParams(
            collective_id=0 if _ABLATE in ("full", "phase1") else None),
    )(x2, Wq, K_ext, V_ext, Wo)

    return out.reshape(B, SQ, DM)


# device time: 20052 ns/iter; 2.3586x vs baseline; 1.0232x over previous
import os

import jax
import jax.numpy as jnp
from jax import lax
from jax.experimental import pallas as pl
from jax.experimental.pallas import tpu as pltpu

_ABLATE = os.environ.get("KERNEL_ABLATE", "full")

N_DEV = 32
B, SQ, SKV, DM = 2, 128, 128, 512
HL, DH = 4, 64
ROWS = B * SQ
CHUNK = ROWS // N_DEV


def kernel(x, Wq, K_ext, V_ext, Wo):
    x2 = x.reshape(ROWS, DM)

    def body(x_ref, wq_ref, k_hbm, v_hbm, wo_ref, out_ref,
             kbuf, vbuf, cbuf, pbuf, p1buf, gbuf,
             local_sems, send1, recv1, send2, recv2):
        me = lax.axis_index("i")

        if _ABLATE == "min":
            out_ref[...] = x_ref[...]
            return

        h0 = me * HL
        if _ABLATE not in ("nodma", "qproj"):
            kcp = pltpu.make_async_copy(
                k_hbm.at[:, :, pl.ds(h0, HL), :], kbuf, local_sems.at[0])
            vcp = pltpu.make_async_copy(
                v_hbm.at[:, :, pl.ds(h0, HL), :], vbuf, local_sems.at[1])
            kcp.start()
            vcp.start()

        q = lax.dot_general(
            x_ref[...].astype(jnp.bfloat16), wq_ref[...].astype(jnp.bfloat16),
            (((1,), (0,)), ((), ())), preferred_element_type=jnp.float32)

        if _ABLATE not in ("nodma", "qproj"):
            kcp.wait()
            vcp.wait()
        kv_k = kbuf[...]
        kv_v = vbuf[...]

        for b in range(B if _ABLATE != "qproj" else 0):
            for h in range(HL):
                qbh = q[b * SQ:(b + 1) * SQ, h * DH:(h + 1) * DH]
                kbh = kv_k[b, :, h, :]
                s = lax.dot_general(
                    qbh.astype(jnp.bfloat16), kbh.astype(jnp.bfloat16),
                    (((1,), (1,)), ((), ())),
                    preferred_element_type=jnp.float32) * 0.125
                mx = jnp.max(s, axis=1, keepdims=True)
                w = jnp.exp(s - mx)
                w = w / jnp.sum(w, axis=1, keepdims=True)
                c = lax.dot_general(
                    w.astype(jnp.bfloat16), kv_v[b, :, h, :].astype(jnp.bfloat16),
                    (((1,), (0,)), ((), ())),
                    preferred_element_type=jnp.float32)
                cbuf[b * SQ:(b + 1) * SQ, h * DH:(h + 1) * DH] = c

        ctx = q if _ABLATE == "qproj" else cbuf[...]
        partial = lax.dot_general(
            ctx.astype(jnp.bfloat16), wo_ref[...].astype(jnp.bfloat16),
            (((1,), (0,)), ((), ())), preferred_element_type=jnp.float32)
        pbuf[...] = partial.reshape(N_DEV, CHUNK, DM).astype(jnp.bfloat16)

        if _ABLATE in ("compute", "nodma", "qproj"):
            out_ref[...] = pbuf[...].astype(jnp.float32).reshape(ROWS, DM)
            return

        bsem = pltpu.get_barrier_semaphore()
        for off in range(1, N_DEV):
            pl.semaphore_signal(
                bsem, inc=1, device_id=((me + off) % N_DEV,),
                device_id_type=pl.DeviceIdType.MESH)
        pl.semaphore_wait(bsem, N_DEV - 1)

        sends1 = []
        for off in range(1, N_DEV):
            dst = (me + off) % N_DEV
            r = pltpu.make_async_remote_copy(
                src_ref=pbuf.at[dst],
                dst_ref=p1buf.at[me],
                send_sem=send1.at[off - 1],
                recv_sem=recv1.at[me],
                device_id=(dst,), device_id_type=pl.DeviceIdType.MESH)
            r.start()
            sends1.append(r)

        acc = pbuf[me].astype(jnp.float32)
        for off in range(1, N_DEV):
            src = (me + N_DEV - off) % N_DEV
            rv = pltpu.make_async_remote_copy(
                src_ref=pbuf.at[0],
                dst_ref=p1buf.at[src],
                send_sem=send1.at[off - 1],
                recv_sem=recv1.at[src],
                device_id=(src,), device_id_type=pl.DeviceIdType.MESH)
            rv.wait_recv()
            acc = acc + p1buf[src].astype(jnp.float32)

        gbuf[me] = acc.astype(jnp.bfloat16)

        if _ABLATE == "phase1":
            for r in sends1:
                r.wait_send()
            out_ref[...] = gbuf[...].astype(jnp.float32).reshape(ROWS, DM)
            return

        sends2 = []
        for off in range(1, N_DEV):
            dst = (me + off) % N_DEV
            r = pltpu.make_async_remote_copy(
                src_ref=gbuf.at[me],
                dst_ref=gbuf.at[me],
                send_sem=send2.at[off - 1],
                recv_sem=recv2.at[me],
                device_id=(dst,), device_id_type=pl.DeviceIdType.MESH)
            r.start()
            sends2.append(r)

        for off in range(1, N_DEV):
            src = (me + N_DEV - off) % N_DEV
            rv = pltpu.make_async_remote_copy(
                src_ref=gbuf.at[0],
                dst_ref=gbuf.at[src],
                send_sem=send2.at[off - 1],
                recv_sem=recv2.at[src],
                device_id=(src,), device_id_type=pl.DeviceIdType.MESH)
            rv.wait_recv()

        for r in sends1 + sends2:
            r.wait_send()

        out_ref[...] = gbuf[...].astype(jnp.float32).reshape(ROWS, DM)

    out = pl.pallas_call(
        body,
        out_shape=jax.ShapeDtypeStruct((ROWS, DM), jnp.float32),
        in_specs=[
            pl.BlockSpec(memory_space=pltpu.VMEM),
            pl.BlockSpec(memory_space=pltpu.VMEM),
            pl.BlockSpec(memory_space=pl.ANY),
            pl.BlockSpec(memory_space=pl.ANY),
            pl.BlockSpec(memory_space=pltpu.VMEM),
        ],
        out_specs=pl.BlockSpec(memory_space=pltpu.VMEM),
        scratch_shapes=[
            pltpu.VMEM((B, SKV, HL, DH), jnp.float32),
            pltpu.VMEM((B, SKV, HL, DH), jnp.float32),
            pltpu.VMEM((ROWS, HL * DH), jnp.float32),
            pltpu.VMEM((N_DEV, CHUNK, DM), jnp.bfloat16),
            pltpu.VMEM((N_DEV, CHUNK, DM), jnp.bfloat16),
            pltpu.VMEM((N_DEV, CHUNK, DM), jnp.bfloat16),
            pltpu.SemaphoreType.DMA((2,)),
            pltpu.SemaphoreType.DMA((N_DEV - 1,)),
            pltpu.SemaphoreType.DMA((N_DEV,)),
            pltpu.SemaphoreType.DMA((N_DEV - 1,)),
            pltpu.SemaphoreType.DMA((N_DEV,)),
        ],
        compiler_params=pltpu.CompilerParams(
            collective_id=0 if _ABLATE in ("full", "phase1") else None),
    )(x2, Wq, K_ext, V_ext, Wo)

    return out.reshape(B, SQ, DM)
